# baseline (speedup 1.0000x reference)
import functools

import jax
import jax.numpy as jnp
from jax import lax
from jax.experimental import pallas as pl
from jax.experimental.pallas import tpu as pltpu
from jax.experimental.pallas import tpu_sc as plsc

N = 10000
E = 320000
D = 128

NC = 2
NS = 16
L = 16
NW = NC * NS
RPW = 320
LAST = N - RPW * (NW - 1)
WIN = 512
JUNK = RPW
SLAB = RPW + 8
K = 128
NB = 3
NQ = 4
RP_PAD = 10440
COL_PAD = E + K


def _mm_body(x_ref, w_ref, o_ref):
    o_ref[...] = jnp.dot(x_ref[...], w_ref[...],
                         preferred_element_type=jnp.float32)


def _matmul(X, W):
    M, BM = X.shape[0], 400
    return pl.pallas_call(
        _mm_body,
        grid=(M // BM,),
        in_specs=[pl.BlockSpec((BM, D), lambda i: (i, 0)),
                  pl.BlockSpec((D, D), lambda i: (0, 0))],
        out_specs=pl.BlockSpec((BM, D), lambda i: (i, 0)),
        out_shape=jax.ShapeDtypeStruct((M, D), jnp.float32),
    )(X, W)


def _sc_body(xp_hbm, col_hbm, rp_hbm, out_hbm,
             win_v, idx_v, seg_v, rows_v, part_sh, gsem, isem, ssem):
    c = lax.axis_index("c")
    s = lax.axis_index("s")
    wid = c * NS + s
    row_base = wid * RPW
    slab = s * SLAB

    pltpu.sync_copy(rp_hbm.at[pl.ds(row_base, WIN)], win_v)

    def _zrow(i, carry):
        for f in range(D // L):
            rows_v[0, i, pl.ds(f * L, L)] = jnp.zeros((L,), jnp.float32)
        return carry
    lax.fori_loop(0, K, _zrow, 0)
    pltpu.sync_copy(rows_v.at[0], part_sh.at[pl.ds(slab, K)])
    pltpu.sync_copy(rows_v.at[0], part_sh.at[pl.ds(slab + K, K)])
    pltpu.sync_copy(rows_v.at[0, pl.ds(0, SLAB - 2 * K)],
                    part_sh.at[pl.ds(slab + 2 * K, SLAB - 2 * K)])

    e_lo = win_v[pl.ds(0, L)][0]
    e_hi = win_v[pl.ds(RPW, L)][0]
    e_al = jnp.bitwise_and(e_lo, -8)
    nchunk = (e_hi - e_al + (K - 1)) // K
    iota = lax.broadcasted_iota(jnp.int32, (L,), 0)

    def _idx_start(j, q):
        pltpu.async_copy(
            col_hbm.at[pl.ds(pl.multiple_of(e_al + j * K, 8), K)],
            idx_v.at[q], isem.at[q])

    def _idx_wait(j, q):
        pltpu.make_async_copy(
            col_hbm.at[pl.ds(pl.multiple_of(e_al + j * K, 8), K)],
            idx_v.at[q], isem.at[q]).wait()

    def _gat_start(q, b):
        pltpu.async_copy(xp_hbm.at[idx_v.at[q]], rows_v.at[b], gsem.at[b])

    def _gat_wait(q, b):
        pltpu.make_async_copy(xp_hbm.at[idx_v.at[q]], rows_v.at[b],
                              gsem.at[b]).wait()

    def _sct_start(b):
        pltpu.async_copy(rows_v.at[b], part_sh.at[seg_v.at[b]], ssem.at[b],
                         add=True)

    def _sct_wait(b):
        pltpu.make_async_copy(rows_v.at[b], part_sh.at[seg_v.at[b]],
                              ssem.at[b]).wait()

    for j in range(NQ):
        @pl.when(j < nchunk)
        def _(j=j):
            _idx_start(j, j)
    for j in range(NB - 1):
        @pl.when(j < nchunk)
        def _(j=j):
            _idx_wait(j, j)
            _gat_start(j, j)

    def _chunk(i, carry):
        b = lax.rem(i, NB)
        q = lax.rem(i, NQ)
        _gat_wait(q, b)

        off = e_al + i * K
        for v in range(K // L):
            evec = off + v * L + iota
            pos = jnp.zeros((L,), jnp.int32)
            bit = WIN // 2
            while bit:
                cand = pos + bit
                val = plsc.load_gather(win_v, [cand])
                pos = jnp.where(val <= evec, cand, pos)
                bit //= 2
            valid = (evec >= e_lo) & (evec < e_hi)
            seg_v[b, pl.ds(v * L, L)] = slab + jnp.where(valid, pos, JUNK)

        _sct_start(b)

        @pl.when(i + NB - 1 < nchunk)
        def _():
            j2 = i + NB - 1
            b2 = lax.rem(j2, NB)

            @pl.when(j2 >= NB)
            def _():
                _sct_wait(b2)
            _idx_wait(j2, lax.rem(j2, NQ))
            _gat_start(lax.rem(j2, NQ), b2)

        @pl.when(i + NQ < nchunk)
        def _():
            j4 = i + NQ
            _idx_start(j4, lax.rem(j4, NQ))

        return carry

    lax.fori_loop(0, nchunk, _chunk, 0)

    for t in range(NB):
        @pl.when(t < nchunk)
        def _(t=t):
            _sct_wait(t)

    @pl.when(wid < NW - 1)
    def _():
        pltpu.sync_copy(part_sh.at[pl.ds(slab, RPW)],
                        out_hbm.at[pl.ds(row_base, RPW)])

    @pl.when(wid == NW - 1)
    def _():
        pltpu.sync_copy(part_sh.at[pl.ds(slab, LAST)],
                        out_hbm.at[pl.ds(row_base, LAST)])


def _sc_spmm(Xp, col_pad, rp_pad):
    mesh = plsc.VectorSubcoreMesh(core_axis_name="c", subcore_axis_name="s")
    k = pl.kernel(
        _sc_body,
        out_type=jax.ShapeDtypeStruct((N, D), jnp.float32),
        mesh=mesh,
        scratch_types=[
            pltpu.VMEM((WIN,), jnp.int32),
            pltpu.VMEM((NQ, K), jnp.int32),
            pltpu.VMEM((NB, K), jnp.int32),
            pltpu.VMEM((NB, K, D), jnp.float32),
            pltpu.VMEM_SHARED((NS * SLAB, D), jnp.float32),
            pltpu.SemaphoreType.DMA((NB,)),
            pltpu.SemaphoreType.DMA((NQ,)),
            pltpu.SemaphoreType.DMA((NB,)),
        ],
        compiler_params=pltpu.CompilerParams(needs_layout_passes=False),
    )
    return k(Xp, col_pad, rp_pad)


def kernel(X, row_pointers, column_index, blockPartition, edgeToColumn,
           edgeToRow, W):
    rp_pad = jnp.full((RP_PAD,), E, dtype=jnp.int32)
    rp_pad = rp_pad.at[:N + 1].set(row_pointers)
    rp_pad = rp_pad.at[0].set(0)
    rp_pad = rp_pad.at[N].set(E)
    col_pad = jnp.concatenate(
        [column_index, jnp.zeros((K,), dtype=jnp.int32)])

    Xp = _matmul(X, W)
    return _sc_spmm(Xp, col_pad, rp_pad)


# speedup vs baseline: 1.0658x; 1.0658x over previous
import functools

import jax
import jax.numpy as jnp
from jax import lax
from jax.experimental import pallas as pl
from jax.experimental.pallas import tpu as pltpu
from jax.experimental.pallas import tpu_sc as plsc

N = 10000
E = 320000
D = 128

NC = 2
NS = 16
L = 16
NW = NC * NS
RPW = 320
LAST = N - RPW * (NW - 1)
WIN = 512
JUNK = RPW
SLAB = RPW + 8
K = 128
NB = 3
NQ = 4
RP_PAD = 10440
EMAXOFF = E - K


def _mm_body(x_ref, w_ref, o_ref):
    o_ref[...] = jnp.dot(x_ref[...], w_ref[...],
                         preferred_element_type=jnp.float32)


def _matmul(X, W):
    M, BM = X.shape[0], 1000
    return pl.pallas_call(
        _mm_body,
        grid=(M // BM,),
        in_specs=[pl.BlockSpec((BM, D), lambda i: (i, 0)),
                  pl.BlockSpec((D, D), lambda i: (0, 0))],
        out_specs=pl.BlockSpec((BM, D), lambda i: (i, 0)),
        out_shape=jax.ShapeDtypeStruct((M, D), jnp.float32),
    )(X, W)


def _sc_body(xp_hbm, col_hbm, rp_hbm, out_hbm,
             win_v, idx_v, seg_v, rows_v, part_sh, gsem, isem, ssem):
    c = lax.axis_index("c")
    s = lax.axis_index("s")
    wid = c * NS + s
    row_base = wid * RPW
    slab = s * SLAB

    pltpu.sync_copy(rp_hbm.at[pl.ds(row_base, WIN)], win_v)

    def _zrow(i, carry):
        for f in range(D // L):
            rows_v[0, i, pl.ds(f * L, L)] = jnp.zeros((L,), jnp.float32)
        return carry
    lax.fori_loop(0, K, _zrow, 0)
    pltpu.sync_copy(rows_v.at[0], part_sh.at[pl.ds(slab, K)])
    pltpu.sync_copy(rows_v.at[0], part_sh.at[pl.ds(slab + K, K)])
    pltpu.sync_copy(rows_v.at[0, pl.ds(0, SLAB - 2 * K)],
                    part_sh.at[pl.ds(slab + 2 * K, SLAB - 2 * K)])

    e_lo = win_v[pl.ds(0, L)][0]
    e_hi = win_v[pl.ds(RPW, L)][0]
    e_al = jnp.bitwise_and(e_lo, -8)
    nchunk = (e_hi - e_al + (K - 1)) // K
    iota = lax.broadcasted_iota(jnp.int32, (L,), 0)

    def _coff(j):
        return pl.multiple_of(e_al + j * K, 8)

    def _idx_start(j, q):
        pltpu.async_copy(col_hbm.at[pl.ds(_coff(j), K)],
                         idx_v.at[q], isem.at[q])

    def _idx_wait(j, q):
        pltpu.make_async_copy(col_hbm.at[pl.ds(_coff(j), K)],
                              idx_v.at[q], isem.at[q]).wait()

    def _gat_start(q, b):
        pltpu.async_copy(xp_hbm.at[idx_v.at[q]], rows_v.at[b], gsem.at[b])

    def _gat_wait(q, b):
        pltpu.make_async_copy(xp_hbm.at[idx_v.at[q]], rows_v.at[b],
                              gsem.at[b]).wait()

    def _sct_start(b):
        pltpu.async_copy(rows_v.at[b], part_sh.at[seg_v.at[b]], ssem.at[b],
                         add=True)

    def _sct_wait(b):
        pltpu.make_async_copy(rows_v.at[b], part_sh.at[seg_v.at[b]],
                              ssem.at[b]).wait()

    for j in range(NQ):
        @pl.when(j < nchunk)
        def _(j=j):
            _idx_start(j, j)
    for j in range(NB - 1):
        @pl.when(j < nchunk)
        def _(j=j):
            _idx_wait(j, j)
            _gat_start(j, j)

    def _chunk(i, carry):
        b = lax.rem(i, NB)
        q = lax.rem(i, NQ)
        _gat_wait(q, b)

        off = e_al + i * K
        for v in range(K // L):
            evec = off + v * L + iota
            pos = jnp.zeros((L,), jnp.int32)
            bit = WIN // 2
            while bit:
                cand = pos + bit
                val = plsc.load_gather(win_v, [cand])
                pos = jnp.where(val <= evec, cand, pos)
                bit //= 2
            valid = (evec >= e_lo) & (evec < e_hi)
            seg_v[b, pl.ds(v * L, L)] = slab + jnp.where(valid, pos, JUNK)

        @pl.when(i >= 1)
        def _():
            _sct_wait(lax.rem(i + NB - 1, NB))
        _sct_start(b)

        @pl.when(i + NB - 1 < nchunk)
        def _():
            j2 = i + NB - 1
            _idx_wait(j2, lax.rem(j2, NQ))
            _gat_start(lax.rem(j2, NQ), lax.rem(j2, NB))

        @pl.when(i + NQ < nchunk)
        def _():
            j4 = i + NQ
            _idx_start(j4, lax.rem(j4, NQ))

        return carry

    lax.fori_loop(0, nchunk, _chunk, 0)

    @pl.when(nchunk >= 1)
    def _():
        _sct_wait(lax.rem(nchunk - 1, NB))

    @pl.when(wid < NW - 1)
    def _():
        pltpu.sync_copy(part_sh.at[pl.ds(slab, RPW)],
                        out_hbm.at[pl.ds(row_base, RPW)])

    @pl.when(wid == NW - 1)
    def _():
        pltpu.sync_copy(part_sh.at[pl.ds(slab, LAST)],
                        out_hbm.at[pl.ds(row_base, LAST)])


def _sc_spmm(Xp, column_index, rp_pad):
    mesh = plsc.VectorSubcoreMesh(core_axis_name="c", subcore_axis_name="s")
    k = pl.kernel(
        _sc_body,
        out_type=jax.ShapeDtypeStruct((N, D), jnp.float32),
        mesh=mesh,
        scratch_types=[
            pltpu.VMEM((WIN,), jnp.int32),
            pltpu.VMEM((NQ, K), jnp.int32),
            pltpu.VMEM((NB, K), jnp.int32),
            pltpu.VMEM((NB, K, D), jnp.float32),
            pltpu.VMEM_SHARED((NS * SLAB, D), jnp.float32),
            pltpu.SemaphoreType.DMA((NB,)),
            pltpu.SemaphoreType.DMA((NQ,)),
            pltpu.SemaphoreType.DMA((NB,)),
        ],
        compiler_params=pltpu.CompilerParams(needs_layout_passes=False),
    )
    return k(Xp, column_index, rp_pad)


def kernel(X, row_pointers, column_index, blockPartition, edgeToColumn,
           edgeToRow, W):
    rp_pad = jnp.concatenate([
        jnp.zeros((1,), jnp.int32),
        lax.slice(row_pointers, (1,), (N,)),
        jnp.full((RP_PAD - N,), E, dtype=jnp.int32),
    ])

    col_pad = jnp.concatenate(
        [column_index, jnp.zeros((K,), dtype=jnp.int32)])
    Xp = _matmul(X, W)
    return _sc_spmm(Xp, col_pad, rp_pad)


# speedup vs baseline: 1.0817x; 1.0149x over previous
import functools

import jax
import jax.numpy as jnp
from jax import lax
from jax.experimental import pallas as pl
from jax.experimental.pallas import tpu as pltpu
from jax.experimental.pallas import tpu_sc as plsc

N = 10000
E = 320000
D = 128

NC = 2
NS = 16
L = 16
NW = NC * NS
RPW = 320
LAST = N - RPW * (NW - 1)
WIN = 512
JUNK = RPW
SLAB = RPW + 8
K = 128
NB = 4
NQ = 6
RP_PAD = 10440
EMAXOFF = E - K


def _mm_body(x_ref, w_ref, o_ref):
    o_ref[...] = jnp.dot(x_ref[...], w_ref[...],
                         preferred_element_type=jnp.float32)


def _matmul(X, W):
    M, BM = X.shape[0], 1000
    return pl.pallas_call(
        _mm_body,
        grid=(M // BM,),
        in_specs=[pl.BlockSpec((BM, D), lambda i: (i, 0)),
                  pl.BlockSpec((D, D), lambda i: (0, 0))],
        out_specs=pl.BlockSpec((BM, D), lambda i: (i, 0)),
        out_shape=jax.ShapeDtypeStruct((M, D), jnp.float32),
    )(X, W)


def _sc_body(xp_hbm, col_hbm, rp_hbm, out_hbm,
             win_v, idx_v, seg_v, rows_v, part_sh, gsem, isem, ssem):
    c = lax.axis_index("c")
    s = lax.axis_index("s")
    wid = c * NS + s
    row_base = wid * RPW
    slab = s * SLAB

    pltpu.sync_copy(rp_hbm.at[pl.ds(row_base, WIN)], win_v)

    def _zrow(i, carry):
        for f in range(D // L):
            rows_v[0, i, pl.ds(f * L, L)] = jnp.zeros((L,), jnp.float32)
        return carry
    lax.fori_loop(0, K, _zrow, 0)
    pltpu.sync_copy(rows_v.at[0], part_sh.at[pl.ds(slab, K)])
    pltpu.sync_copy(rows_v.at[0], part_sh.at[pl.ds(slab + K, K)])
    pltpu.sync_copy(rows_v.at[0, pl.ds(0, SLAB - 2 * K)],
                    part_sh.at[pl.ds(slab + 2 * K, SLAB - 2 * K)])

    e_lo = win_v[pl.ds(0, L)][0]
    e_hi = win_v[pl.ds(RPW, L)][0]
    e_al = jnp.bitwise_and(e_lo, -8)
    nchunk = (e_hi - e_al + (K - 1)) // K
    iota = lax.broadcasted_iota(jnp.int32, (L,), 0)

    def _coff(j):
        return pl.multiple_of(e_al + j * K, 8)

    def _idx_start(j, q):
        pltpu.async_copy(col_hbm.at[pl.ds(_coff(j), K)],
                         idx_v.at[q], isem.at[q])

    def _idx_wait(j, q):
        pltpu.make_async_copy(col_hbm.at[pl.ds(_coff(j), K)],
                              idx_v.at[q], isem.at[q]).wait()

    def _gat_start(q, b):
        pltpu.async_copy(xp_hbm.at[idx_v.at[q]], rows_v.at[b], gsem.at[b])

    def _gat_wait(q, b):
        pltpu.make_async_copy(xp_hbm.at[idx_v.at[q]], rows_v.at[b],
                              gsem.at[b]).wait()

    def _sct_start(b):
        pltpu.async_copy(rows_v.at[b], part_sh.at[seg_v.at[b]], ssem.at[b],
                         add=True)

    def _sct_wait(b):
        pltpu.make_async_copy(rows_v.at[b], part_sh.at[seg_v.at[b]],
                              ssem.at[b]).wait()

    for j in range(NQ):
        @pl.when(j < nchunk)
        def _(j=j):
            _idx_start(j, j)
    for j in range(NB - 1):
        @pl.when(j < nchunk)
        def _(j=j):
            _idx_wait(j, j)
            _gat_start(j, j)

    def _chunk(i, carry):
        b = lax.rem(i, NB)
        q = lax.rem(i, NQ)
        _gat_wait(q, b)

        off = e_al + i * K
        for v in range(K // L):
            evec = off + v * L + iota
            pos = jnp.zeros((L,), jnp.int32)
            bit = WIN // 2
            while bit:
                cand = pos + bit
                val = plsc.load_gather(win_v, [cand])
                pos = jnp.where(val <= evec, cand, pos)
                bit //= 2
            valid = (evec >= e_lo) & (evec < e_hi)
            seg_v[b, pl.ds(v * L, L)] = slab + jnp.where(valid, pos, JUNK)

        @pl.when(i >= 1)
        def _():
            _sct_wait(lax.rem(i + NB - 1, NB))
        _sct_start(b)

        @pl.when(i + NB - 1 < nchunk)
        def _():
            j2 = i + NB - 1
            _idx_wait(j2, lax.rem(j2, NQ))
            _gat_start(lax.rem(j2, NQ), lax.rem(j2, NB))

        @pl.when(i + NQ < nchunk)
        def _():
            j4 = i + NQ
            _idx_start(j4, lax.rem(j4, NQ))

        return carry

    lax.fori_loop(0, nchunk, _chunk, 0)

    @pl.when(nchunk >= 1)
    def _():
        _sct_wait(lax.rem(nchunk - 1, NB))

    @pl.when(wid < NW - 1)
    def _():
        pltpu.sync_copy(part_sh.at[pl.ds(slab, RPW)],
                        out_hbm.at[pl.ds(row_base, RPW)])

    @pl.when(wid == NW - 1)
    def _():
        pltpu.sync_copy(part_sh.at[pl.ds(slab, LAST)],
                        out_hbm.at[pl.ds(row_base, LAST)])


def _sc_spmm(Xp, column_index, rp_pad):
    mesh = plsc.VectorSubcoreMesh(core_axis_name="c", subcore_axis_name="s")
    k = pl.kernel(
        _sc_body,
        out_type=jax.ShapeDtypeStruct((N, D), jnp.float32),
        mesh=mesh,
        scratch_types=[
            pltpu.VMEM((WIN,), jnp.int32),
            pltpu.VMEM((NQ, K), jnp.int32),
            pltpu.VMEM((NB, K), jnp.int32),
            pltpu.VMEM((NB, K, D), jnp.float32),
            pltpu.VMEM_SHARED((NS * SLAB, D), jnp.float32),
            pltpu.SemaphoreType.DMA((NB,)),
            pltpu.SemaphoreType.DMA((NQ,)),
            pltpu.SemaphoreType.DMA((NB,)),
        ],
        compiler_params=pltpu.CompilerParams(needs_layout_passes=False),
    )
    return k(Xp, column_index, rp_pad)


def kernel(X, row_pointers, column_index, blockPartition, edgeToColumn,
           edgeToRow, W):
    rp_pad = jnp.concatenate([
        jnp.zeros((1,), jnp.int32),
        lax.slice(row_pointers, (1,), (N,)),
        jnp.full((RP_PAD - N,), E, dtype=jnp.int32),
    ])

    col_pad = jnp.concatenate(
        [column_index, jnp.zeros((K,), dtype=jnp.int32)])
    Xp = _matmul(X, W)
    return _sc_spmm(Xp, col_pad, rp_pad)
